# baseline (speedup 1.0000x reference)
import functools

import jax
import jax.numpy as jnp
from jax import lax
from jax.experimental import pallas as pl
from jax.experimental.pallas import tpu as pltpu
from jax.experimental.pallas import tpu_sc as plsc

IDX_W = 128
NSLOT = 4
PADW = 128


@functools.lru_cache(maxsize=None)
def _make_gather(n_tot, v, d):
    info = plsc.get_sparse_core_info()
    nw = info.num_cores * info.num_subcores
    n_per_w = n_tot // nw
    t_tot = n_per_w // IDX_W
    assert n_tot % nw == 0 and n_per_w % IDX_W == 0 and t_tot % NSLOT == 0
    mesh = plsc.VectorSubcoreMesh(core_axis_name="c", subcore_axis_name="s")

    @functools.partial(
        pl.kernel,
        mesh=mesh,
        out_type=jax.ShapeDtypeStruct((n_tot // IDX_W, IDX_W, PADW), jnp.float32),
        compiler_params=pltpu.CompilerParams(
            use_tc_tiling_on_sc=True, needs_layout_passes=False
        ),
        scratch_types=[
            pltpu.VMEM((t_tot, IDX_W), jnp.int32),
            pltpu.VMEM((NSLOT, IDX_W, PADW), jnp.float32),
        ]
        + [pltpu.SemaphoreType.DMA] * (2 * NSLOT),
    )
    def gather(xf_hbm, table_hbm, out_hbm, idx_v, rows_v, *sems):
        gsems = sems[:NSLOT]
        ssems = sems[NSLOT:]
        wid = lax.axis_index("s") * info.num_cores + lax.axis_index("c")
        r0 = wid * n_per_w

        pltpu.sync_copy(xf_hbm.at[pl.ds(wid * t_tot, t_tot)], idx_v)

        def fire_gather(t, slot):
            pltpu.async_copy(
                table_hbm.at[idx_v.at[t]], rows_v.at[slot], gsems[slot]
            )

        def wait_gather(t, slot):
            pltpu.make_async_copy(
                table_hbm.at[idx_v.at[t]], rows_v.at[slot], gsems[slot]
            ).wait()

        def out_view(t):
            return out_hbm.at[wid * t_tot + t]

        def fire_store(t, slot):
            pltpu.async_copy(rows_v.at[slot], out_view(t), ssems[slot])

        def wait_store(t, slot):
            pltpu.make_async_copy(rows_v.at[slot], out_view(t), ssems[slot]).wait()

        for slot in range(NSLOT):
            fire_gather(slot, slot)

        def body(r, carry):
            t0 = r * NSLOT
            for slot in range(NSLOT):
                t = t0 + slot
                wait_gather(t, slot)
                fire_store(t, slot)
            for slot in range(NSLOT):
                t = t0 + slot
                wait_store(t, slot)
                fire_gather(t + NSLOT, slot)
            return carry

        lax.fori_loop(0, t_tot // NSLOT - 1, body, 0, unroll=False)

        t0 = t_tot - NSLOT
        for slot in range(NSLOT):
            wait_gather(t0 + slot, slot)
            fire_store(t0 + slot, slot)
        for slot in range(NSLOT):
            wait_store(t0 + slot, slot)

    return gather


def kernel(x, table):
    b, h = x.shape
    v, d = table.shape
    xf = x.reshape(b * h // IDX_W, IDX_W).astype(jnp.int32)
    tp = jnp.pad(table, ((0, 0), (0, PADW - d)))
    out = _make_gather(b * h, v, d)(xf, tp)
    return out.reshape(b * h, PADW)[:, :d].reshape(b, h, d)


# speedup vs baseline: 1.1335x; 1.1335x over previous
import functools

import jax
import jax.numpy as jnp
from jax import lax
from jax.experimental import pallas as pl
from jax.experimental.pallas import tpu as pltpu
from jax.experimental.pallas import tpu_sc as plsc

IDX_W = 128
NSLOT = 4


@functools.lru_cache(maxsize=None)
def _make_gather(n_tot, v, d):
    info = plsc.get_sparse_core_info()
    nw = info.num_cores * info.num_subcores
    n_per_w = n_tot // nw
    t_tot = n_per_w // IDX_W
    assert n_tot % nw == 0 and n_per_w % IDX_W == 0 and t_tot % NSLOT == 0
    mesh = plsc.VectorSubcoreMesh(core_axis_name="c", subcore_axis_name="s")

    @functools.partial(
        pl.kernel,
        mesh=mesh,
        out_type=jax.ShapeDtypeStruct((n_tot, d), jnp.float32),
        compiler_params=pltpu.CompilerParams(
            use_tc_tiling_on_sc=False, needs_layout_passes=False
        ),
        scratch_types=[
            pltpu.VMEM((t_tot, IDX_W), jnp.int32),
            pltpu.VMEM((NSLOT, IDX_W, d), jnp.float32),
        ]
        + [pltpu.SemaphoreType.DMA] * (2 * NSLOT),
    )
    def gather(xf_hbm, table_hbm, out_hbm, idx_v, rows_v, *sems):
        gsems = sems[:NSLOT]
        ssems = sems[NSLOT:]
        wid = lax.axis_index("s") * info.num_cores + lax.axis_index("c")
        r0 = wid * n_per_w

        pltpu.sync_copy(xf_hbm.at[pl.ds(wid * t_tot, t_tot)], idx_v)

        def fire_gather(t, slot):
            pltpu.async_copy(
                table_hbm.at[idx_v.at[t]], rows_v.at[slot], gsems[slot]
            )

        def wait_gather(t, slot):
            pltpu.make_async_copy(
                table_hbm.at[idx_v.at[t]], rows_v.at[slot], gsems[slot]
            ).wait()

        def out_view(t):
            return out_hbm.at[pl.ds(r0 + t * IDX_W, IDX_W)]

        def fire_store(t, slot):
            pltpu.async_copy(rows_v.at[slot], out_view(t), ssems[slot])

        def wait_store(t, slot):
            pltpu.make_async_copy(rows_v.at[slot], out_view(t), ssems[slot]).wait()

        for slot in range(NSLOT):
            fire_gather(slot, slot)

        def body(r, carry):
            t0 = r * NSLOT
            for slot in range(NSLOT):
                t = t0 + slot
                wait_gather(t, slot)
                fire_store(t, slot)
            for slot in range(NSLOT):
                t = t0 + slot
                wait_store(t, slot)
                fire_gather(t + NSLOT, slot)
            return carry

        lax.fori_loop(0, t_tot // NSLOT - 1, body, 0, unroll=False)

        t0 = t_tot - NSLOT
        for slot in range(NSLOT):
            wait_gather(t0 + slot, slot)
            fire_store(t0 + slot, slot)
        for slot in range(NSLOT):
            wait_store(t0 + slot, slot)

    return gather


def kernel(x, table):
    b, h = x.shape
    v, d = table.shape
    xf = x.reshape(b * h // IDX_W, IDX_W).astype(jnp.int32)
    out = _make_gather(b * h, v, d)(xf, table)
    return out.reshape(b, h, d)


# speedup vs baseline: 1.1343x; 1.0007x over previous
import functools

import jax
import jax.numpy as jnp
from jax import lax
from jax.experimental import pallas as pl
from jax.experimental.pallas import tpu as pltpu
from jax.experimental.pallas import tpu_sc as plsc

IDX_W = 128
NSLOT = 8


@functools.lru_cache(maxsize=None)
def _make_gather(n_tot, v, d):
    info = plsc.get_sparse_core_info()
    nw = info.num_cores * info.num_subcores
    n_per_w = n_tot // nw
    t_tot = n_per_w // IDX_W
    assert n_tot % nw == 0 and n_per_w % IDX_W == 0 and t_tot % NSLOT == 0
    mesh = plsc.VectorSubcoreMesh(core_axis_name="c", subcore_axis_name="s")

    @functools.partial(
        pl.kernel,
        mesh=mesh,
        out_type=jax.ShapeDtypeStruct((n_tot, d), jnp.float32),
        compiler_params=pltpu.CompilerParams(
            use_tc_tiling_on_sc=False, needs_layout_passes=False
        ),
        scratch_types=[
            pltpu.VMEM((t_tot, IDX_W), jnp.int32),
            pltpu.VMEM((NSLOT, IDX_W, d), jnp.float32),
        ]
        + [pltpu.SemaphoreType.DMA] * (2 * NSLOT),
    )
    def gather(xf_hbm, table_hbm, out_hbm, idx_v, rows_v, *sems):
        gsems = sems[:NSLOT]
        ssems = sems[NSLOT:]
        wid = lax.axis_index("s") * info.num_cores + lax.axis_index("c")
        r0 = wid * n_per_w

        pltpu.sync_copy(xf_hbm.at[pl.ds(wid * t_tot, t_tot)], idx_v)

        def fire_gather(t, slot):
            pltpu.async_copy(
                table_hbm.at[idx_v.at[t]], rows_v.at[slot], gsems[slot]
            )

        def wait_gather(t, slot):
            pltpu.make_async_copy(
                table_hbm.at[idx_v.at[t]], rows_v.at[slot], gsems[slot]
            ).wait()

        def out_view(t):
            return out_hbm.at[pl.ds(r0 + t * IDX_W, IDX_W)]

        def fire_store(t, slot):
            pltpu.async_copy(rows_v.at[slot], out_view(t), ssems[slot])

        def wait_store(t, slot):
            pltpu.make_async_copy(rows_v.at[slot], out_view(t), ssems[slot]).wait()

        for slot in range(NSLOT):
            fire_gather(slot, slot)

        def body(r, carry):
            t0 = r * NSLOT
            for slot in range(NSLOT):
                t = t0 + slot
                wait_gather(t, slot)
                fire_store(t, slot)
            for slot in range(NSLOT):
                t = t0 + slot
                wait_store(t, slot)
                fire_gather(t + NSLOT, slot)
            return carry

        lax.fori_loop(0, t_tot // NSLOT - 1, body, 0, unroll=False)

        t0 = t_tot - NSLOT
        for slot in range(NSLOT):
            wait_gather(t0 + slot, slot)
            fire_store(t0 + slot, slot)
        for slot in range(NSLOT):
            wait_store(t0 + slot, slot)

    return gather


def kernel(x, table):
    b, h = x.shape
    v, d = table.shape
    xf = x.reshape(b * h // IDX_W, IDX_W).astype(jnp.int32)
    out = _make_gather(b * h, v, d)(xf, table)
    return out.reshape(b, h, d)
